# baseline (speedup 1.0000x reference)
import functools

import jax
import jax.numpy as jnp
from jax import lax
from jax.experimental import pallas as pl
from jax.experimental.pallas import tpu as pltpu
from jax.experimental.pallas import tpu_sc as plsc

_NUM_EMB = 1000000
_EMB_DIM = 64
_SUB_DIM = 32
_BASE_SEED = 42

_NC = 2
_NS = 16
_NW = _NC * _NS
_CHUNK = 128
_LANE = 128
_FPAD = 32


def _hash16(h, seed):
    h = h ^ jnp.uint32(seed)
    h = h * jnp.uint32(2654435761)
    h = h ^ (h >> jnp.uint32(16))
    h = h * jnp.uint32(2246822519)
    h = h ^ (h >> jnp.uint32(13))
    hf = h.astype(jnp.float32)
    q = (hf * jnp.float32(1.0 / _NUM_EMB)).astype(jnp.uint32)
    r = (h - q * jnp.uint32(_NUM_EMB)).astype(jnp.int32)
    m = jnp.int32(_NUM_EMB)
    r = jnp.where(r < 0, r + m, r)
    r = jnp.where(r < 0, r + m, r)
    r = jnp.where(r >= m, r - m, r)
    r = jnp.where(r >= m, r - m, r)
    return r


def _make_sc_kernel(b0):
    rows_per_w = b0 // _NW
    lookups_per_w = rows_per_w * _FPAD
    n_chunks = lookups_per_w // _CHUNK
    n_out_rows = b0 * _FPAD * (_LANE // _SUB_DIM)
    mesh = plsc.VectorSubcoreMesh(core_axis_name="c", subcore_axis_name="s")

    n_blk = 4
    blk = n_chunks // n_blk
    assert blk * n_blk == n_chunks

    @functools.partial(
        pl.kernel,
        mesh=mesh,
        compiler_params=pltpu.CompilerParams(use_tc_tiling_on_sc=False),
        out_type=jax.ShapeDtypeStruct((n_out_rows, _SUB_DIM), jnp.float32),
        scratch_types=[
            pltpu.VMEM((rows_per_w, _LANE), jnp.int32),
            pltpu.VMEM((n_chunks, _CHUNK), jnp.int32),
            pltpu.VMEM((n_chunks, _CHUNK), jnp.int32),
            pltpu.VMEM((n_chunks, _CHUNK), jnp.int32),
            pltpu.VMEM((n_chunks, _CHUNK), jnp.int32),
            pltpu.VMEM((blk, _CHUNK, _SUB_DIM), jnp.float32),
            pltpu.VMEM((blk, _CHUNK, _SUB_DIM), jnp.float32),
            pltpu.SemaphoreType.DMA,
            pltpu.SemaphoreType.DMA,
        ],
    )
    def k(ids_hbm, w0_hbm, w1_hbm, out_hbm, ids_v, idx0_v, idx1_v,
          oe_v, oo_v, rows0_v, rows1_v, sem_g, sem_s):
        wid = lax.axis_index("s") * _NC + lax.axis_index("c")
        row0 = wid * rows_per_w
        base = row0 * _FPAD
        pltpu.sync_copy(ids_hbm.at[pl.ds(row0, rows_per_w), :], ids_v)
        lanes = lax.iota(jnp.int32, 16)

        rows_per_chunk = _CHUNK // _FPAD

        def hash_chunk(j, _):
            for rr in range(rows_per_chunk):
                for c in range(_FPAD // 16):
                    p = rr * _FPAD + c * 16
                    raw = ids_v[j * rows_per_chunk + rr,
                                pl.ds(c * 16, 16)].astype(jnp.uint32)
                    idx0_v[j, pl.ds(p, 16)] = _hash16(raw, _BASE_SEED)
                    idx1_v[j, pl.ds(p, 16)] = _hash16(raw, _BASE_SEED + 1)
                    oe = 4 * (base + j * _CHUNK + p + lanes)
                    oe_v[j, pl.ds(p, 16)] = oe
                    oo_v[j, pl.ds(p, 16)] = oe + 1
            return 0

        lax.fori_loop(0, n_chunks, hash_chunk, 0)

        for b in range(n_blk):
            gathers = []
            for j in range(b * blk, (b + 1) * blk):
                s = j - b * blk
                gathers.append(pltpu.async_copy(
                    w0_hbm.at[idx0_v.at[j]], rows0_v.at[s], sem_g))
                gathers.append(pltpu.async_copy(
                    w1_hbm.at[idx1_v.at[j]], rows1_v.at[s], sem_g))
            for g in gathers:
                g.wait()
            scatters = []
            for j in range(b * blk, (b + 1) * blk):
                s = j - b * blk
                scatters.append(pltpu.async_copy(
                    rows0_v.at[s], out_hbm.at[oe_v.at[j]], sem_s))
                scatters.append(pltpu.async_copy(
                    rows1_v.at[s], out_hbm.at[oo_v.at[j]], sem_s))
            for sc in scatters:
                sc.wait()

    return k


def kernel(input_ids, W0, W1):
    shape = input_ids.shape
    b0, fields = shape
    idsp = jnp.pad(input_ids, ((0, 0), (0, _LANE - fields)))
    out2 = _make_sc_kernel(b0)(idsp, W0, W1)
    out3 = out2.reshape(b0, _FPAD, _LANE)[:, :fields, :_EMB_DIM]
    return out3.reshape(shape + (_EMB_DIM,))


# speedup vs baseline: 1.3063x; 1.3063x over previous
import functools

import jax
import jax.numpy as jnp
from jax import lax
from jax.experimental import pallas as pl
from jax.experimental.pallas import tpu as pltpu
from jax.experimental.pallas import tpu_sc as plsc

_NUM_EMB = 1000000
_EMB_DIM = 64
_SUB_DIM = 32
_BASE_SEED = 42

_NC = 2
_NS = 16
_NW = _NC * _NS
_CHUNK = 128


def _hash16(h, seed):
    h = h ^ jnp.uint32(seed)
    h = h * jnp.uint32(2654435761)
    h = h ^ (h >> jnp.uint32(16))
    h = h * jnp.uint32(2246822519)
    h = h ^ (h >> jnp.uint32(13))
    hf = h.astype(jnp.float32)
    q = (hf * jnp.float32(1.0 / _NUM_EMB)).astype(jnp.uint32)
    r = (h - q * jnp.uint32(_NUM_EMB)).astype(jnp.int32)
    m = jnp.int32(_NUM_EMB)
    r = jnp.where(r < 0, r + m, r)
    r = jnp.where(r < 0, r + m, r)
    r = jnp.where(r >= m, r - m, r)
    r = jnp.where(r >= m, r - m, r)
    return r


def _make_sc_kernel(batch, fields, fpad):
    assert batch % (_NW * _CHUNK) == 0
    b_per_w = batch // _NW
    n_chunks = b_per_w // _CHUNK
    n_out_rows = (batch // fields) * fpad * (128 // _SUB_DIM)
    inv_f = 1.0 / fields
    mesh = plsc.VectorSubcoreMesh(core_axis_name="c", subcore_axis_name="s")

    n_blk = 2
    blk = n_chunks // n_blk
    assert blk * n_blk == n_chunks

    @functools.partial(
        pl.kernel,
        mesh=mesh,
        compiler_params=pltpu.CompilerParams(use_tc_tiling_on_sc=False),
        out_type=jax.ShapeDtypeStruct((n_out_rows, _SUB_DIM), jnp.float32),
        scratch_types=[
            pltpu.VMEM((b_per_w,), jnp.int32),
            pltpu.VMEM((n_chunks, _CHUNK), jnp.int32),
            pltpu.VMEM((n_chunks, _CHUNK), jnp.int32),
            pltpu.VMEM((n_chunks, _CHUNK), jnp.int32),
            pltpu.VMEM((n_chunks, _CHUNK), jnp.int32),
            pltpu.VMEM((blk, _CHUNK, _SUB_DIM), jnp.float32),
            pltpu.VMEM((blk, _CHUNK, _SUB_DIM), jnp.float32),
            pltpu.SemaphoreType.DMA,
            pltpu.SemaphoreType.DMA,
        ],
    )
    def k(ids_hbm, w0_hbm, w1_hbm, out_hbm, ids_v, idx0_v, idx1_v,
          oe_v, oo_v, rows0_v, rows1_v, sem_g, sem_s):
        wid = lax.axis_index("s") * _NC + lax.axis_index("c")
        base = wid * b_per_w
        pltpu.sync_copy(ids_hbm.at[pl.ds(base, b_per_w)], ids_v)
        lanes = lax.iota(jnp.int32, 16)

        def hash_chunk(j, _):
            for i in range(_CHUNK // 16):
                raw = ids_v[pl.ds(j * _CHUNK + i * 16, 16)].astype(jnp.uint32)
                idx0_v[j, pl.ds(i * 16, 16)] = _hash16(raw, _BASE_SEED)
                idx1_v[j, pl.ds(i * 16, 16)] = _hash16(raw, _BASE_SEED + 1)
                g = base + j * _CHUNK + i * 16 + lanes
                bq = (g.astype(jnp.float32) * jnp.float32(inv_f)
                      + jnp.float32(0.001)).astype(jnp.int32)
                fr = g - bq * fields
                oe = 4 * (bq * fpad + fr)
                oe_v[j, pl.ds(i * 16, 16)] = oe
                oo_v[j, pl.ds(i * 16, 16)] = oe + 1
            return 0

        lax.fori_loop(0, n_chunks, hash_chunk, 0)

        for b in range(n_blk):
            gathers = []
            for j in range(b * blk, (b + 1) * blk):
                s = j - b * blk
                gathers.append(pltpu.async_copy(
                    w0_hbm.at[idx0_v.at[j]], rows0_v.at[s], sem_g))
                gathers.append(pltpu.async_copy(
                    w1_hbm.at[idx1_v.at[j]], rows1_v.at[s], sem_g))
            for g in gathers:
                g.wait()
            scatters = []
            for j in range(b * blk, (b + 1) * blk):
                s = j - b * blk
                scatters.append(pltpu.async_copy(
                    rows0_v.at[s], out_hbm.at[oe_v.at[j]], sem_s))
                scatters.append(pltpu.async_copy(
                    rows1_v.at[s], out_hbm.at[oo_v.at[j]], sem_s))
            for sc in scatters:
                sc.wait()

    return k


def kernel(input_ids, W0, W1):
    shape = input_ids.shape
    b0, fields = shape
    fpad = -(-fields // 8) * 8
    flat = input_ids.reshape(-1)
    out2 = _make_sc_kernel(flat.shape[0], fields, fpad)(flat, W0, W1)
    out3 = out2.reshape(b0, fpad, 128)[:, :fields, :_EMB_DIM]
    return out3.reshape(shape + (_EMB_DIM,))
